# baseline (speedup 1.0000x reference)
import functools

import jax
import jax.numpy as jnp
from jax import lax
from jax.experimental import pallas as pl
from jax.experimental.pallas import tpu as pltpu
from jax.experimental.pallas import tpu_sc as plsc

_N = 10000
_E = 320000
_F = 128
_G = 16
_EPS = 1e-5

_NC = 2
_NS = 16
_NW = _NC * _NS
_CH = 128
_NCHUNK = 80
_EPT = _NCHUNK * _CH
_EPAD = _EPT * _NW
_NPAD = 10112
_STRIPE = _NPAD // _NS

_sc_mesh = plsc.VectorSubcoreMesh(core_axis_name="c", subcore_axis_name="s",
                                  num_cores=_NC, num_subcores=_NS)
_sc_params = pltpu.CompilerParams(needs_layout_passes=False)


@functools.partial(
    pl.kernel,
    out_type=jax.ShapeDtypeStruct((_NW, _NPAD), jnp.float32),
    mesh=_sc_mesh,
    scratch_types=[pltpu.VMEM((_NCHUNK, _CH), jnp.int32),
                   pltpu.VMEM((_NPAD,), jnp.float32)],
    compiler_params=_sc_params)
def _sc_degree(dst_hbm, out_hbm, dst_v, deg_v):
    cid = lax.axis_index("c")
    sid = lax.axis_index("s")
    wid = sid * _NC + cid
    pltpu.sync_copy(dst_hbm.at[wid], dst_v)
    zeros = jnp.zeros((16,), jnp.float32)

    @pl.loop(0, _NPAD, step=16)
    def _(i):
        deg_v[pl.ds(i, 16)] = zeros

    ones = jnp.ones((16,), jnp.float32)

    @pl.loop(0, _NCHUNK)
    def _(j):
        for k in range(_CH // 16):
            idx = dst_v[j, pl.ds(k * 16, 16)]
            plsc.addupdate_scatter(deg_v, [idx], ones)

    pltpu.sync_copy(deg_v, out_hbm.at[wid])


_NBUF = 2
_NHALF = _NCHUNK // 2


@functools.partial(
    pl.kernel,
    out_type=jax.ShapeDtypeStruct((_NC, _NPAD, _F), jnp.float32),
    mesh=_sc_mesh,
    scratch_types=[pltpu.VMEM((_NHALF, _CH), jnp.int32),
                   pltpu.VMEM((_NHALF, _CH), jnp.int32),
                   pltpu.VMEM((_CH, _F), jnp.float32),
                   pltpu.VMEM((_CH, _F), jnp.float32),
                   pltpu.SemaphoreType.DMA,
                   pltpu.SemaphoreType.DMA,
                   pltpu.VMEM_SHARED((_NPAD, _F), jnp.float32)],
    compiler_params=_sc_params)
def _sc_scatter(hs_hbm, src_hbm, dst_hbm, zeros_hbm, out_hbm,
                src_v, dst_v, b0, b1, s0, s1, acc_sh):
    bufs = (b0, b1)
    sems = (s0, s1)
    cid = lax.axis_index("c")
    sid = lax.axis_index("s")
    wid = sid * _NC + cid
    row0 = sid * _STRIPE
    pltpu.sync_copy(zeros_hbm.at[pl.ds(row0, _STRIPE)],
                    acc_sh.at[pl.ds(row0, _STRIPE)])
    plsc.subcore_barrier()

    for half in range(2):
        base = half * _NHALF
        pltpu.sync_copy(src_hbm.at[wid].at[pl.ds(base, _NHALF)], src_v)
        pltpu.sync_copy(dst_hbm.at[wid].at[pl.ds(base, _NHALF)], dst_v)
        for b in range(_NBUF):
            pltpu.async_copy(hs_hbm.at[src_v.at[b]], bufs[b], sems[b])

        @pl.loop(0, _NHALF - _NBUF, step=_NBUF)
        def _(j):
            for b in range(_NBUF):
                jj = j + b
                pltpu.make_async_copy(hs_hbm.at[src_v.at[0]],
                                      bufs[b], sems[b]).wait()
                pltpu.sync_copy(bufs[b], acc_sh.at[dst_v.at[jj]], add=True)
                pltpu.async_copy(hs_hbm.at[src_v.at[jj + _NBUF]], bufs[b],
                                 sems[b])

        for b in range(_NBUF):
            pltpu.make_async_copy(hs_hbm.at[src_v.at[0]], bufs[b],
                                  sems[b]).wait()
            pltpu.sync_copy(bufs[b],
                            acc_sh.at[dst_v.at[_NHALF - _NBUF + b]],
                            add=True)

    plsc.subcore_barrier()
    pltpu.sync_copy(acc_sh.at[pl.ds(row0, _STRIPE)],
                    out_hbm.at[cid].at[pl.ds(row0, _STRIPE)])


_BLK = 1264
_NBLK = _NPAD // _BLK

_row_spec = pl.BlockSpec((_BLK, _F), lambda i: (i, 0))
_col_spec = pl.BlockSpec((_BLK, 1), lambda i: (i, 0))
_w_spec = pl.BlockSpec((_F, _F), lambda i: (0, 0))
_vec_spec = pl.BlockSpec((1, _F), lambda i: (0, 0))
_st_spec = pl.BlockSpec((2, _F), lambda i: (0, 0))


def _tc_mm1_body(x_ref, w_ref, out_ref):
    out_ref[...] = jnp.dot(x_ref[...], w_ref[...],
                           preferred_element_type=jnp.float32)


_tc_mm1 = pl.pallas_call(
    _tc_mm1_body,
    grid=(_NBLK,),
    in_specs=[_row_spec, _w_spec],
    out_specs=_row_spec,
    out_shape=jax.ShapeDtypeStruct((_NPAD, _F), jnp.float32))


def _tc_dinv_body(degp_ref, dinv_ref):
    deg = jnp.sum(degp_ref[...], axis=0).reshape(_NPAD, 1) + 1.0
    rows = lax.broadcasted_iota(jnp.int32, (_NPAD, 1), 0)
    dinv_ref[...] = jnp.where(rows < _N, lax.rsqrt(deg), 0.0)


_tc_dinv = pl.pallas_call(
    _tc_dinv_body,
    out_shape=jax.ShapeDtypeStruct((_NPAD, 1), jnp.float32))


def _tc_scale_body(h_ref, dinv_ref, out_ref):
    pid = pl.program_id(0)
    rows = lax.broadcasted_iota(jnp.int32, (_BLK, 1), 0) + pid * _BLK
    out_ref[...] = jnp.where(rows < _N, dinv_ref[...] * h_ref[...], 0.0)


_tc_scale = pl.pallas_call(
    _tc_scale_body,
    grid=(_NBLK,),
    in_specs=[_row_spec, _col_spec],
    out_specs=_row_spec,
    out_shape=jax.ShapeDtypeStruct((_NPAD, _F), jnp.float32))


def _tc_post_body(p_ref, hs_ref, dinv_ref, b_ref, act_ref, st_ref):
    pid = pl.program_id(0)
    rows = lax.broadcasted_iota(jnp.int32, (_BLK, 1), 0) + pid * _BLK
    mask = rows < _N
    agg = p_ref[0] + p_ref[1] + hs_ref[...]
    pre = dinv_ref[...] * agg + b_ref[...]
    act = jnp.where(pre > 0, pre, 0.01 * pre)
    act = jnp.where(mask, act, 0.0)
    act_ref[...] = act
    st = jnp.concatenate(
        [jnp.sum(act, axis=0, keepdims=True),
         jnp.sum(act * act, axis=0, keepdims=True)], axis=0)

    @pl.when(pid == 0)
    def _():
        st_ref[...] = st

    @pl.when(pid > 0)
    def _():
        st_ref[...] += st


_tc_post = pl.pallas_call(
    _tc_post_body,
    grid=(_NBLK,),
    in_specs=[pl.BlockSpec((2, _BLK, _F), lambda i: (0, i, 0)),
              _row_spec, _col_spec, _vec_spec],
    out_specs=(_row_spec, _st_spec),
    out_shape=(jax.ShapeDtypeStruct((_NPAD, _F), jnp.float32),
               jax.ShapeDtypeStruct((2, _F), jnp.float32)))


def _tc_matmul_body(act_ref, st_ref, g_ref, be_ref, w_ref, dinv_ref,
                    out_ref):
    mu = st_ref[0:1, :] * (1.0 / _N)
    var = st_ref[1:2, :] * (1.0 / _N) - mu * mu
    a = g_ref[...] * lax.rsqrt(var + _EPS)
    c = be_ref[...] - mu * a
    bn = act_ref[...] * a + c
    h = jnp.dot(bn, w_ref[...], preferred_element_type=jnp.float32)
    out_ref[...] = dinv_ref[...] * h


_tc_matmul = pl.pallas_call(
    _tc_matmul_body,
    grid=(_NBLK,),
    in_specs=[_row_spec, _st_spec, _vec_spec, _vec_spec, _w_spec, _col_spec],
    out_specs=_row_spec,
    out_shape=jax.ShapeDtypeStruct((_NPAD, _F), jnp.float32))


def _tc_pool_body(act_ref, st_ref, g_ref, be_ref, batch_ref, out_ref,
                  acc_s, cnt_s):
    pid = pl.program_id(0)
    mu = st_ref[0:1, :] * (1.0 / _N)
    var = st_ref[1:2, :] * (1.0 / _N) - mu * mu
    a = g_ref[...] * lax.rsqrt(var + _EPS)
    c = be_ref[...] - mu * a
    bn = act_ref[...] * a + c
    seg = lax.broadcasted_iota(jnp.int32, (_G, _BLK), 0)
    onehot = (batch_ref[...].reshape(1, _BLK) == seg).astype(jnp.float32)
    ps = jnp.dot(onehot, bn, preferred_element_type=jnp.float32)
    pc = jnp.sum(onehot, axis=1, keepdims=True)

    @pl.when(pid == 0)
    def _():
        acc_s[...] = ps
        cnt_s[...] = pc

    @pl.when(pid > 0)
    def _():
        acc_s[...] += ps
        cnt_s[...] += pc

    @pl.when(pid == _NBLK - 1)
    def _():
        out_ref[...] = acc_s[...] / jnp.maximum(cnt_s[...], 1.0)


_tc_pool = pl.pallas_call(
    _tc_pool_body,
    grid=(_NBLK,),
    in_specs=[_row_spec, _st_spec, _vec_spec, _vec_spec,
              pl.BlockSpec((1, 1, _BLK), lambda i: (i, 0, 0))],
    out_specs=pl.BlockSpec((_G, _F), lambda i: (0, 0)),
    out_shape=jax.ShapeDtypeStruct((_G, _F), jnp.float32),
    scratch_shapes=[pltpu.VMEM((_G, _F), jnp.float32),
                    pltpu.VMEM((_G, 1), jnp.float32)])


def kernel(x, W1, b1, g1, be1, W2, b2, g2, be2, W3, b3, g3, be3,
           edge_index, batch):
    src = edge_index[0].astype(jnp.int32)
    dst = edge_index[1].astype(jnp.int32)
    perw = _E // _NW
    padw = _EPT - perw
    pad = _N + (jnp.arange(_NW * padw, dtype=jnp.int32) % (_NPAD - _N))
    pad = pad.reshape(_NW, padw)
    srcp = jnp.concatenate([src.reshape(_NW, perw), pad],
                           axis=1).reshape(_NW, _NCHUNK, _CH)
    dstp = jnp.concatenate([dst.reshape(_NW, perw), pad],
                           axis=1).reshape(_NW, _NCHUNK, _CH)
    zeros = jnp.zeros((_NPAD, _F), jnp.float32)

    batchp = jnp.concatenate([batch.astype(jnp.int32),
                              jnp.full((_NPAD - _N,), _G, jnp.int32)])
    batchp = batchp.reshape(_NBLK, 1, _BLK)

    degp = _sc_degree(dstp)
    h1 = _tc_mm1(x, W1)
    dinv = _tc_dinv(degp)
    hs = _tc_scale(h1, dinv)
    p = _sc_scatter(hs, srcp, dstp, zeros)
    act, st = _tc_post(p, hs, dinv, b1.reshape(1, _F))
    hs = _tc_matmul(act, st, g1.reshape(1, _F), be1.reshape(1, _F), W2, dinv)
    p = _sc_scatter(hs, srcp, dstp, zeros)
    act, st = _tc_post(p, hs, dinv, b2.reshape(1, _F))
    hs = _tc_matmul(act, st, g2.reshape(1, _F), be2.reshape(1, _F), W3, dinv)
    p = _sc_scatter(hs, srcp, dstp, zeros)
    act, st = _tc_post(p, hs, dinv, b3.reshape(1, _F))
    return _tc_pool(act, st, g3.reshape(1, _F), be3.reshape(1, _F), batchp)


# speedup vs baseline: 1.0350x; 1.0350x over previous
import functools

import jax
import jax.numpy as jnp
from jax import lax
from jax.experimental import pallas as pl
from jax.experimental.pallas import tpu as pltpu
from jax.experimental.pallas import tpu_sc as plsc

_N = 10000
_E = 320000
_F = 128
_G = 16
_EPS = 1e-5

_NC = 2
_NS = 16
_NW = _NC * _NS
_CH = 128
_NCHUNK = 80
_EPT = _NCHUNK * _CH
_EPAD = _EPT * _NW
_NPAD = 10112
_STRIPE = _NPAD // _NS

_sc_mesh = plsc.VectorSubcoreMesh(core_axis_name="c", subcore_axis_name="s",
                                  num_cores=_NC, num_subcores=_NS)
_sc_params = pltpu.CompilerParams(needs_layout_passes=False)


@functools.partial(
    pl.kernel,
    out_type=jax.ShapeDtypeStruct((_NW, _NPAD), jnp.float32),
    mesh=_sc_mesh,
    scratch_types=[pltpu.VMEM((_NCHUNK, _CH), jnp.int32),
                   pltpu.VMEM((_NPAD,), jnp.float32)],
    compiler_params=_sc_params)
def _sc_degree(dst_hbm, out_hbm, dst_v, deg_v):
    cid = lax.axis_index("c")
    sid = lax.axis_index("s")
    wid = sid * _NC + cid
    pltpu.sync_copy(dst_hbm.at[wid], dst_v)
    zeros = jnp.zeros((16,), jnp.float32)

    @pl.loop(0, _NPAD, step=16)
    def _(i):
        deg_v[pl.ds(i, 16)] = zeros

    ones = jnp.ones((16,), jnp.float32)

    @pl.loop(0, _NCHUNK)
    def _(j):
        for k in range(_CH // 16):
            idx = dst_v[j, pl.ds(k * 16, 16)]
            plsc.addupdate_scatter(deg_v, [idx], ones)

    pltpu.sync_copy(deg_v, out_hbm.at[wid])


_NBUF = 2
_NHALF = _NCHUNK // 2


@functools.partial(
    pl.kernel,
    out_type=jax.ShapeDtypeStruct((_NC, _NPAD, _F), jnp.float32),
    mesh=_sc_mesh,
    scratch_types=[pltpu.VMEM((_NHALF, _CH), jnp.int32),
                   pltpu.VMEM((_NHALF, _CH), jnp.int32),
                   pltpu.VMEM((_CH, _F), jnp.float32),
                   pltpu.VMEM((_CH, _F), jnp.float32),
                   pltpu.SemaphoreType.DMA,
                   pltpu.SemaphoreType.DMA,
                   pltpu.VMEM_SHARED((_NPAD, _F), jnp.float32)],
    compiler_params=_sc_params)
def _sc_scatter(hs_hbm, src_hbm, dst_hbm, zeros_hbm, out_hbm,
                src_v, dst_v, b0, b1, s0, s1, acc_sh):
    bufs = (b0, b1)
    sems = (s0, s1)
    cid = lax.axis_index("c")
    sid = lax.axis_index("s")
    wid = sid * _NC + cid
    row0 = sid * _STRIPE
    pltpu.sync_copy(zeros_hbm.at[pl.ds(row0, _STRIPE)],
                    acc_sh.at[pl.ds(row0, _STRIPE)])
    plsc.subcore_barrier()

    for half in range(2):
        base = half * _NHALF
        pltpu.sync_copy(src_hbm.at[wid].at[pl.ds(base, _NHALF)], src_v)
        pltpu.sync_copy(dst_hbm.at[wid].at[pl.ds(base, _NHALF)], dst_v)
        for b in range(_NBUF):
            pltpu.async_copy(hs_hbm.at[src_v.at[b]], bufs[b], sems[b])

        @pl.loop(0, _NHALF - _NBUF, step=_NBUF)
        def _(j):
            for b in range(_NBUF):
                jj = j + b
                pltpu.make_async_copy(hs_hbm.at[src_v.at[0]],
                                      bufs[b], sems[b]).wait()
                pltpu.sync_copy(bufs[b], acc_sh.at[dst_v.at[jj]], add=True)
                pltpu.async_copy(hs_hbm.at[src_v.at[jj + _NBUF]], bufs[b],
                                 sems[b])

        for b in range(_NBUF):
            pltpu.make_async_copy(hs_hbm.at[src_v.at[0]], bufs[b],
                                  sems[b]).wait()
            pltpu.sync_copy(bufs[b],
                            acc_sh.at[dst_v.at[_NHALF - _NBUF + b]],
                            add=True)

    plsc.subcore_barrier()
    pltpu.sync_copy(acc_sh.at[pl.ds(row0, _STRIPE)],
                    out_hbm.at[cid].at[pl.ds(row0, _STRIPE)])


_BLK = 1264
_NBLK = _NPAD // _BLK

_row_spec = pl.BlockSpec((_BLK, _F), lambda i: (i, 0))
_col_spec = pl.BlockSpec((_BLK, 1), lambda i: (i, 0))
_w_spec = pl.BlockSpec((_F, _F), lambda i: (0, 0))
_vec_spec = pl.BlockSpec((1, _F), lambda i: (0, 0))
_st_spec = pl.BlockSpec((2, _F), lambda i: (0, 0))


def _tc_pre_body(degp_ref, x_ref, w_ref, dinv_ref, hs_ref):
    deg = jnp.sum(degp_ref[...], axis=0).reshape(_NPAD, 1) + 1.0
    rows = lax.broadcasted_iota(jnp.int32, (_NPAD, 1), 0)
    dinv = jnp.where(rows < _N, lax.rsqrt(deg), 0.0)
    dinv_ref[...] = dinv
    h = jnp.dot(x_ref[...], w_ref[...], preferred_element_type=jnp.float32)
    hs_ref[0:_N, :] = dinv[0:_N, :] * h
    hs_ref[_N:_NPAD, :] = jnp.zeros((_NPAD - _N, _F), jnp.float32)


_tc_pre = pl.pallas_call(
    _tc_pre_body,
    out_shape=(jax.ShapeDtypeStruct((_NPAD, 1), jnp.float32),
               jax.ShapeDtypeStruct((_NPAD, _F), jnp.float32)))


def _leaky_bn_phase0(p_ref, hs_ref, dinv_ref, b_ref, act_s, st_s, i):
    rows = lax.broadcasted_iota(jnp.int32, (_BLK, 1), 0) + i * _BLK
    mask = rows < _N
    agg = p_ref[0] + p_ref[1] + hs_ref[...]
    pre = dinv_ref[...] * agg + b_ref[...]
    act = jnp.where(pre > 0, pre, 0.01 * pre)
    act = jnp.where(mask, act, 0.0)
    act_s[pl.ds(i * _BLK, _BLK), :] = act
    st = jnp.concatenate(
        [jnp.sum(act, axis=0, keepdims=True),
         jnp.sum(act * act, axis=0, keepdims=True)], axis=0)

    @pl.when(i == 0)
    def _():
        st_s[...] = st

    @pl.when(i > 0)
    def _():
        st_s[...] += st


def _bn_affine(st_s, g_ref, be_ref):
    mu = st_s[0:1, :] * (1.0 / _N)
    var = st_s[1:2, :] * (1.0 / _N) - mu * mu
    a = g_ref[...] * lax.rsqrt(var + _EPS)
    c = be_ref[...] - mu * a
    return a, c


def _tc_mid_body(p_ref, hs_ref, dinv_ref, b_ref, g_ref, be_ref, w_ref,
                 out_ref, act_s, st_s):
    ph = pl.program_id(0)
    i = pl.program_id(1)

    @pl.when(ph == 0)
    def _():
        _leaky_bn_phase0(p_ref, hs_ref, dinv_ref, b_ref, act_s, st_s, i)

    @pl.when(ph == 1)
    def _():
        a, c = _bn_affine(st_s, g_ref, be_ref)
        bn = act_s[pl.ds(i * _BLK, _BLK), :] * a + c
        h = jnp.dot(bn, w_ref[...], preferred_element_type=jnp.float32)
        out_ref[...] = dinv_ref[...] * h


_tc_mid = pl.pallas_call(
    _tc_mid_body,
    grid=(2, _NBLK),
    in_specs=[pl.BlockSpec((2, _BLK, _F), lambda ph, i: (0, i * (1 - ph), 0)),
              pl.BlockSpec((_BLK, _F), lambda ph, i: (i * (1 - ph), 0)),
              pl.BlockSpec((_BLK, 1), lambda ph, i: (i, 0)),
              pl.BlockSpec((1, _F), lambda ph, i: (0, 0)),
              pl.BlockSpec((1, _F), lambda ph, i: (0, 0)),
              pl.BlockSpec((1, _F), lambda ph, i: (0, 0)),
              pl.BlockSpec((_F, _F), lambda ph, i: (0, 0))],
    out_specs=pl.BlockSpec((_BLK, _F), lambda ph, i: (i * ph, 0)),
    out_shape=jax.ShapeDtypeStruct((_NPAD, _F), jnp.float32),
    scratch_shapes=[pltpu.VMEM((_NPAD, _F), jnp.float32),
                    pltpu.VMEM((2, _F), jnp.float32)])


def _tc_fin_body(p_ref, hs_ref, dinv_ref, b_ref, g_ref, be_ref, batch_ref,
                 out_ref, act_s, st_s, acc_s, cnt_s):
    ph = pl.program_id(0)
    i = pl.program_id(1)

    @pl.when(ph == 0)
    def _():
        _leaky_bn_phase0(p_ref, hs_ref, dinv_ref, b_ref, act_s, st_s, i)

    @pl.when(ph == 1)
    def _():
        a, c = _bn_affine(st_s, g_ref, be_ref)
        bn = act_s[pl.ds(i * _BLK, _BLK), :] * a + c
        seg = lax.broadcasted_iota(jnp.int32, (_G, _BLK), 0)
        onehot = (batch_ref[...].reshape(1, _BLK) == seg).astype(jnp.float32)
        ps = jnp.dot(onehot, bn, preferred_element_type=jnp.float32)
        pc = jnp.sum(onehot, axis=1, keepdims=True)

        @pl.when(i == 0)
        def _():
            acc_s[...] = ps
            cnt_s[...] = pc

        @pl.when(i > 0)
        def _():
            acc_s[...] += ps
            cnt_s[...] += pc

        @pl.when(i == _NBLK - 1)
        def _():
            out_ref[...] = acc_s[...] / jnp.maximum(cnt_s[...], 1.0)


_tc_fin = pl.pallas_call(
    _tc_fin_body,
    grid=(2, _NBLK),
    in_specs=[pl.BlockSpec((2, _BLK, _F), lambda ph, i: (0, i * (1 - ph), 0)),
              pl.BlockSpec((_BLK, _F), lambda ph, i: (i * (1 - ph), 0)),
              pl.BlockSpec((_BLK, 1), lambda ph, i: (i * (1 - ph), 0)),
              pl.BlockSpec((1, _F), lambda ph, i: (0, 0)),
              pl.BlockSpec((1, _F), lambda ph, i: (0, 0)),
              pl.BlockSpec((1, _F), lambda ph, i: (0, 0)),
              pl.BlockSpec((1, 1, _BLK), lambda ph, i: (i * ph, 0, 0))],
    out_specs=pl.BlockSpec((_G, _F), lambda ph, i: (0, 0)),
    out_shape=jax.ShapeDtypeStruct((_G, _F), jnp.float32),
    scratch_shapes=[pltpu.VMEM((_NPAD, _F), jnp.float32),
                    pltpu.VMEM((2, _F), jnp.float32),
                    pltpu.VMEM((_G, _F), jnp.float32),
                    pltpu.VMEM((_G, 1), jnp.float32)])


def kernel(x, W1, b1, g1, be1, W2, b2, g2, be2, W3, b3, g3, be3,
           edge_index, batch):
    src = edge_index[0].astype(jnp.int32)
    dst = edge_index[1].astype(jnp.int32)
    perw = _E // _NW
    padw = _EPT - perw
    pad = _N + (jnp.arange(_NW * padw, dtype=jnp.int32) % (_NPAD - _N))
    pad = pad.reshape(_NW, padw)
    srcp = jnp.concatenate([src.reshape(_NW, perw), pad],
                           axis=1).reshape(_NW, _NCHUNK, _CH)
    dstp = jnp.concatenate([dst.reshape(_NW, perw), pad],
                           axis=1).reshape(_NW, _NCHUNK, _CH)
    zeros = jnp.zeros((_NPAD, _F), jnp.float32)

    batchp = jnp.concatenate([batch.astype(jnp.int32),
                              jnp.full((_NPAD - _N,), _G, jnp.int32)])
    batchp = batchp.reshape(_NBLK, 1, _BLK)

    degp = _sc_degree(dstp)
    dinv, hs = _tc_pre(degp, x, W1)
    p = _sc_scatter(hs, srcp, dstp, zeros)
    hs = _tc_mid(p, hs, dinv, b1.reshape(1, _F), g1.reshape(1, _F),
                 be1.reshape(1, _F), W2)
    p = _sc_scatter(hs, srcp, dstp, zeros)
    hs = _tc_mid(p, hs, dinv, b2.reshape(1, _F), g2.reshape(1, _F),
                 be2.reshape(1, _F), W3)
    p = _sc_scatter(hs, srcp, dstp, zeros)
    return _tc_fin(p, hs, dinv, b3.reshape(1, _F), g3.reshape(1, _F),
                   be3.reshape(1, _F), batchp)


# speedup vs baseline: 1.0781x; 1.0416x over previous
import functools

import jax
import jax.numpy as jnp
from jax import lax
from jax.experimental import pallas as pl
from jax.experimental.pallas import tpu as pltpu
from jax.experimental.pallas import tpu_sc as plsc

_N = 10000
_E = 320000
_F = 128
_G = 16
_EPS = 1e-5

_NC = 2
_NS = 16
_NW = _NC * _NS
_CH = 128
_NCHUNK = 80
_EPT = _NCHUNK * _CH
_EPAD = _EPT * _NW
_NPAD = 10112
_STRIPE = _NPAD // _NS

_sc_mesh = plsc.VectorSubcoreMesh(core_axis_name="c", subcore_axis_name="s",
                                  num_cores=_NC, num_subcores=_NS)
_sc_params = pltpu.CompilerParams(needs_layout_passes=False)


@functools.partial(
    pl.kernel,
    out_type=jax.ShapeDtypeStruct((_NW, _NPAD), jnp.float32),
    mesh=_sc_mesh,
    scratch_types=[pltpu.VMEM((_NCHUNK, _CH), jnp.int32),
                   pltpu.VMEM((_NPAD,), jnp.float32)],
    compiler_params=_sc_params)
def _sc_degree(dst_hbm, out_hbm, dst_v, deg_v):
    cid = lax.axis_index("c")
    sid = lax.axis_index("s")
    wid = sid * _NC + cid
    pltpu.sync_copy(dst_hbm.at[wid], dst_v)
    zeros = jnp.zeros((16,), jnp.float32)

    @pl.loop(0, _NPAD, step=16)
    def _(i):
        deg_v[pl.ds(i, 16)] = zeros

    ones = jnp.ones((16,), jnp.float32)

    @pl.loop(0, _NCHUNK)
    def _(j):
        for k in range(_CH // 16):
            idx = dst_v[j, pl.ds(k * 16, 16)]
            plsc.addupdate_scatter(deg_v, [idx], ones)

    pltpu.sync_copy(deg_v, out_hbm.at[wid])


_NBUF = 2
_NHALF = _NCHUNK // 2


@functools.partial(
    pl.kernel,
    out_type=jax.ShapeDtypeStruct((_NC, _NPAD, _F), jnp.float32),
    mesh=_sc_mesh,
    scratch_types=[pltpu.VMEM((_NHALF, _CH), jnp.int32),
                   pltpu.VMEM((_NHALF, _CH), jnp.int32),
                   pltpu.VMEM((_CH, _F), jnp.float32),
                   pltpu.VMEM((_CH, _F), jnp.float32),
                   pltpu.SemaphoreType.DMA,
                   pltpu.SemaphoreType.DMA,
                   pltpu.VMEM_SHARED((_NPAD, _F), jnp.float32)],
    compiler_params=_sc_params)
def _sc_scatter(hs_hbm, src_hbm, dst_hbm, zeros_hbm, out_hbm,
                src_v, dst_v, b0, b1, s0, s1, acc_sh):
    bufs = (b0, b1)
    sems = (s0, s1)
    cid = lax.axis_index("c")
    sid = lax.axis_index("s")
    wid = sid * _NC + cid
    row0 = sid * _STRIPE
    zeroed = False

    for half in range(2):
        base = half * _NHALF
        pltpu.sync_copy(src_hbm.at[wid].at[pl.ds(base, _NHALF)], src_v)
        pltpu.sync_copy(dst_hbm.at[wid].at[pl.ds(base, _NHALF)], dst_v)
        for b in range(_NBUF):
            pltpu.async_copy(hs_hbm.at[src_v.at[b]], bufs[b], sems[b])
        if not zeroed:
            pltpu.sync_copy(zeros_hbm.at[pl.ds(row0, _STRIPE)],
                            acc_sh.at[pl.ds(row0, _STRIPE)])
            plsc.subcore_barrier()
            zeroed = True

        @pl.loop(0, _NHALF - _NBUF, step=_NBUF)
        def _(j):
            for b in range(_NBUF):
                jj = j + b
                pltpu.make_async_copy(hs_hbm.at[src_v.at[0]],
                                      bufs[b], sems[b]).wait()
                pltpu.sync_copy(bufs[b], acc_sh.at[dst_v.at[jj]], add=True)
                pltpu.async_copy(hs_hbm.at[src_v.at[jj + _NBUF]], bufs[b],
                                 sems[b])

        for b in range(_NBUF):
            pltpu.make_async_copy(hs_hbm.at[src_v.at[0]], bufs[b],
                                  sems[b]).wait()
            pltpu.sync_copy(bufs[b],
                            acc_sh.at[dst_v.at[_NHALF - _NBUF + b]],
                            add=True)

    plsc.subcore_barrier()
    pltpu.sync_copy(acc_sh.at[pl.ds(row0, _STRIPE)],
                    out_hbm.at[cid].at[pl.ds(row0, _STRIPE)])


def _tc_pre_body(degp_ref, x_ref, w_ref, dinv_ref, hs_ref):
    deg = jnp.sum(degp_ref[...], axis=0).reshape(_NPAD, 1) + 1.0
    rows = lax.broadcasted_iota(jnp.int32, (_NPAD, 1), 0)
    dinv = jnp.where(rows < _N, lax.rsqrt(deg), 0.0)
    dinv_ref[...] = dinv
    h = jnp.dot(x_ref[...], w_ref[...], preferred_element_type=jnp.float32)
    hs_ref[0:_N, :] = dinv[0:_N, :] * h
    hs_ref[_N:_NPAD, :] = jnp.zeros((_NPAD - _N, _F), jnp.float32)


_tc_pre = pl.pallas_call(
    _tc_pre_body,
    out_shape=(jax.ShapeDtypeStruct((_NPAD, 1), jnp.float32),
               jax.ShapeDtypeStruct((_NPAD, _F), jnp.float32)))


def _tc_mid_body(p_ref, hs_ref, dinv_ref, b_ref, g_ref, be_ref, w_ref,
                 out_ref):
    dinv = dinv_ref[0:_N, :]
    agg = p_ref[0, 0:_N, :] + p_ref[1, 0:_N, :] + hs_ref[0:_N, :]
    pre = dinv * agg + b_ref[...]
    act = jnp.where(pre > 0, pre, 0.01 * pre)
    mu = jnp.mean(act, axis=0, keepdims=True)
    cen = act - mu
    var = jnp.mean(cen * cen, axis=0, keepdims=True)
    bn = cen * (g_ref[...] * lax.rsqrt(var + _EPS)) + be_ref[...]
    h = jnp.dot(bn, w_ref[...], preferred_element_type=jnp.float32)
    out_ref[0:_N, :] = dinv * h
    out_ref[_N:_NPAD, :] = jnp.zeros((_NPAD - _N, _F), jnp.float32)


_tc_mid = pl.pallas_call(
    _tc_mid_body,
    out_shape=jax.ShapeDtypeStruct((_NPAD, _F), jnp.float32))


def _tc_fin_body(p_ref, hs_ref, dinv_ref, b_ref, g_ref, be_ref, batch_ref,
                 out_ref):
    dinv = dinv_ref[0:_N, :]
    agg = p_ref[0, 0:_N, :] + p_ref[1, 0:_N, :] + hs_ref[0:_N, :]
    pre = dinv * agg + b_ref[...]
    act = jnp.where(pre > 0, pre, 0.01 * pre)
    mu = jnp.mean(act, axis=0, keepdims=True)
    cen = act - mu
    var = jnp.mean(cen * cen, axis=0, keepdims=True)
    bn = cen * (g_ref[...] * lax.rsqrt(var + _EPS)) + be_ref[...]
    seg = lax.broadcasted_iota(jnp.int32, (_G, _N), 0)
    onehot = (batch_ref[...].reshape(1, _N) == seg).astype(jnp.float32)
    sums = jnp.dot(onehot, bn, preferred_element_type=jnp.float32)
    cnt = jnp.sum(onehot, axis=1, keepdims=True)
    out_ref[...] = sums / jnp.maximum(cnt, 1.0)


_tc_fin = pl.pallas_call(
    _tc_fin_body,
    out_shape=jax.ShapeDtypeStruct((_G, _F), jnp.float32))


def kernel(x, W1, b1, g1, be1, W2, b2, g2, be2, W3, b3, g3, be3,
           edge_index, batch):
    src = edge_index[0].astype(jnp.int32)
    dst = edge_index[1].astype(jnp.int32)
    perw = _E // _NW
    padw = _EPT - perw
    pad = _N + (jnp.arange(_NW * padw, dtype=jnp.int32) % (_NPAD - _N))
    pad = pad.reshape(_NW, padw)
    srcp = jnp.concatenate([src.reshape(_NW, perw), pad],
                           axis=1).reshape(_NW, _NCHUNK, _CH)
    dstp = jnp.concatenate([dst.reshape(_NW, perw), pad],
                           axis=1).reshape(_NW, _NCHUNK, _CH)
    zeros = jnp.zeros((_NPAD, _F), jnp.float32)

    batchp = batch.astype(jnp.int32).reshape(1, _N)

    degp = _sc_degree(dstp)
    dinv, hs = _tc_pre(degp, x, W1)
    p = _sc_scatter(hs, srcp, dstp, zeros)
    hs = _tc_mid(p, hs, dinv, b1.reshape(1, _F), g1.reshape(1, _F),
                 be1.reshape(1, _F), W2)
    p = _sc_scatter(hs, srcp, dstp, zeros)
    hs = _tc_mid(p, hs, dinv, b2.reshape(1, _F), g2.reshape(1, _F),
                 be2.reshape(1, _F), W3)
    p = _sc_scatter(hs, srcp, dstp, zeros)
    return _tc_fin(p, hs, dinv, b3.reshape(1, _F), g3.reshape(1, _F),
                   be3.reshape(1, _F), batchp)
